# baseline (speedup 1.0000x reference)
import functools

import jax
import jax.numpy as jnp
from jax import lax
from jax.experimental import pallas as pl
from jax.experimental.pallas import tpu as pltpu
from jax.experimental.pallas import tpu_sc as plsc

_L = 16


def _bcast(vec, idx):
    dnums = lax.GatherDimensionNumbers(
        offset_dims=(), collapsed_slice_dims=(0,), start_index_map=(0,))
    return lax.gather(vec, idx[:, None], dnums, slice_sizes=(1,),
                      mode=lax.GatherScatterMode.PROMISE_IN_BOUNDS)


def _make_relayout(V, D, NW):
    G = V // 128
    mesh = plsc.VectorSubcoreMesh(core_axis_name="c", subcore_axis_name="s")
    out_t = jax.ShapeDtypeStruct((V // 2, 2 * D), jnp.float32)

    @functools.partial(
        pl.kernel,
        out_type=(out_t, out_t, out_t),
        mesh=mesh,
        scratch_types=[
            pltpu.VMEM((D, 128), jnp.float32),
            pltpu.VMEM((D, 2 * D), jnp.float32),
            pltpu.SemaphoreType.DMA,
        ],
        compiler_params=pltpu.CompilerParams(
            use_tc_tiling_on_sc=True, needs_layout_passes=False),
    )
    def relayout(wn_t, wz_t, wp_t, tn, tz, tp, on, oz, op, slab_v, blk_v,
                 sem):
        wid = lax.axis_index("s") * 2 + lax.axis_index("c")
        per = G // NW
        rem = G - per * NW
        start = wid * per + jnp.minimum(wid, rem)
        count = per + jnp.where(wid < rem, 1, 0)
        lane = lax.iota(jnp.int32, _L)

        def do_granule(g, carry):
            for src, dst in ((wn_t, on), (wz_t, oz), (wp_t, op)):
                pltpu.sync_copy(src.at[:, pl.ds(g * 128, 128)], slab_v)

                def tr_f(f, carry2):
                    for m in range(128 // _L):
                        v = slab_v[f, pl.ds(m * _L, _L)]
                        flat = (m * _L + lane) * D + f
                        row = lax.shift_right_logical(flat, 7)
                        col = lax.bitwise_and(flat, 2 * D - 1)
                        plsc.store_scatter(blk_v, [row, col], v)
                    return carry2

                lax.fori_loop(0, D, tr_f, 0)
                pltpu.sync_copy(blk_v, dst.at[pl.ds(g * D, D)])
            return carry

        lax.fori_loop(start, start + count, do_granule, 0)

        Vt = V - G * 128
        if Vt:
            nrow = Vt * D // (2 * D)

            @pl.when(wid == NW - 1)
            def _tail():
                for src, dst in ((tn, on), (tz, oz), (tp, op)):
                    pltpu.sync_copy(src, dst.at[pl.ds(G * D, nrow)])

    return relayout


def _make_gather(N, V, D, C, NW):
    n_per_w = N // NW
    n_chunks = n_per_w // C
    mesh = plsc.VectorSubcoreMesh(core_axis_name="c", subcore_axis_name="s")

    @functools.partial(
        pl.kernel,
        out_type=jax.ShapeDtypeStruct((N, D), jnp.float32),
        mesh=mesh,
        scratch_types=[
            pltpu.VMEM((C,), jnp.int32),
            pltpu.VMEM((C,), jnp.float32),
            pltpu.VMEM((C,), jnp.float32),
            pltpu.VMEM((C,), jnp.float32),
            pltpu.VMEM((C, D), jnp.float32),
            pltpu.VMEM((C, D), jnp.float32),
            pltpu.VMEM((C, D), jnp.float32),
            pltpu.VMEM((C, D), jnp.float32),
            pltpu.SemaphoreType.DMA,
        ],
        compiler_params=pltpu.CompilerParams(use_tc_tiling_on_sc=False),
    )
    def sc_kernel(ids_hbm, wn_hbm, wz_hbm, wp_hbm, s0_hbm, s1_hbm, s2_hbm,
                  out_hbm, idx_v, p0_v, p1_v, p2_v, en_v, ez_v, ep_v, o_v,
                  sem):
        wid = lax.axis_index("s") * 2 + lax.axis_index("c")
        w_base = wid * n_per_w

        def chunk_body(c, carry):
            base = w_base + c * C
            pltpu.sync_copy(ids_hbm.at[pl.ds(base, C)], idx_v)
            cp0 = pltpu.async_copy(s0_hbm.at[idx_v], p0_v, sem)
            cp1 = pltpu.async_copy(s1_hbm.at[idx_v], p1_v, sem)
            cp2 = pltpu.async_copy(s2_hbm.at[idx_v], p2_v, sem)
            cp3 = pltpu.async_copy(wn_hbm.at[idx_v], en_v, sem)
            cp4 = pltpu.async_copy(wz_hbm.at[idx_v], ez_v, sem)
            cp5 = pltpu.async_copy(wp_hbm.at[idx_v], ep_v, sem)
            cp0.wait()
            cp1.wait()
            cp2.wait()
            cp3.wait()
            cp4.wait()
            cp5.wait()

            for i in range(C // _L):
                sl = pl.ds(i * _L, _L)
                l0 = p0_v[sl]
                l1 = p1_v[sl]
                l2 = p2_v[sl]
                m = jnp.maximum(jnp.maximum(l0, l1), l2)
                e0 = jnp.exp(l0 - m)
                e1 = jnp.exp(l1 - m)
                e2 = jnp.exp(l2 - m)
                inv = 1.0 / (e0 + e1 + e2)
                p0_v[sl] = e0 * inv
                p1_v[sl] = e1 * inv
                p2_v[sl] = e2 * inv

            def grp_body(g, carry):
                gsl = pl.ds(g * _L, _L)
                pv0 = p0_v[gsl]
                pv1 = p1_v[gsl]
                pv2 = p2_v[gsl]
                for t in range(_L):
                    j = g * _L + t
                    ts = jnp.full((_L,), t, jnp.int32)
                    pb0 = _bcast(pv0, ts)
                    pb1 = _bcast(pv1, ts)
                    pb2 = _bcast(pv2, ts)
                    for d in range(D // _L):
                        sl = pl.ds(d * _L, _L)
                        o_v[j, sl] = (pb0 * en_v[j, sl] + pb1 * ez_v[j, sl]
                                      + pb2 * ep_v[j, sl])
                return carry

            lax.fori_loop(0, C // _L, grp_body, 0)
            pltpu.sync_copy(o_v, out_hbm.at[pl.ds(base, C)])
            return carry

        lax.fori_loop(0, n_chunks, chunk_body, 0)

    return sc_kernel


@jax.jit
def kernel(input_ids, W_neg1, W_zero, W_pos1, sup_w):
    B, S = input_ids.shape
    V, D = W_neg1.shape
    N = B * S
    NW = 32
    C = 128
    ids_flat = input_ids.reshape(N).astype(jnp.int32)
    s0, s1, s2 = [sup_w[:, i] for i in range(3)]
    wn_t = W_neg1.T
    wz_t = W_zero.T
    wp_t = W_pos1.T
    Gfull = (V // 128) * 128
    nrow = (V - Gfull) * D // (2 * D)
    tn = W_neg1[Gfull:].reshape(nrow, 2 * D)
    tz = W_zero[Gfull:].reshape(nrow, 2 * D)
    tp = W_pos1[Gfull:].reshape(nrow, 2 * D)
    rn, rz, rp = _make_relayout(V, D, NW)(wn_t, wz_t, wp_t, tn, tz, tp)
    wn = rn.reshape(V, D)
    wz = rz.reshape(V, D)
    wp = rp.reshape(V, D)
    out = _make_gather(N, V, D, C, NW)(ids_flat, wn, wz, wp, s0, s1, s2)
    return out.reshape(B, S, D)


# speedup vs baseline: 1.1473x; 1.1473x over previous
import functools

import jax
import jax.numpy as jnp
from jax import lax
from jax.experimental import pallas as pl
from jax.experimental.pallas import tpu as pltpu
from jax.experimental.pallas import tpu_sc as plsc

_L = 16


def _bcast(vec, idx):
    dnums = lax.GatherDimensionNumbers(
        offset_dims=(), collapsed_slice_dims=(0,), start_index_map=(0,))
    return lax.gather(vec, idx[:, None], dnums, slice_sizes=(1,),
                      mode=lax.GatherScatterMode.PROMISE_IN_BOUNDS)


def _make_relayout(V, D, NW):
    G = V // 128
    K = 2
    W128 = K * 128
    BR = K * D
    B = G // K
    mesh = plsc.VectorSubcoreMesh(core_axis_name="c", subcore_axis_name="s")
    out_t = jax.ShapeDtypeStruct((V // 2, 2 * D), jnp.float32)
    slab_t = pltpu.VMEM((D, W128), jnp.float32)

    @functools.partial(
        pl.kernel,
        out_type=(out_t, out_t, out_t),
        mesh=mesh,
        scratch_types=[
            slab_t, slab_t, slab_t,
            slab_t, slab_t, slab_t,
            pltpu.VMEM((BR, 2 * D), jnp.float32),
            pltpu.SemaphoreType.DMA,
            pltpu.SemaphoreType.DMA,
        ],
        compiler_params=pltpu.CompilerParams(
            use_tc_tiling_on_sc=True, needs_layout_passes=False),
    )
    def relayout(wn_t, wz_t, wp_t, tn, tz, tp, on, oz, op,
                 s00, s01, s02, s10, s11, s12, blk_v, sem0, sem1):
        wid = lax.axis_index("s") * 2 + lax.axis_index("c")
        per = B // NW
        rem = B - per * NW
        start = wid * per + jnp.minimum(wid, rem)
        count = per + jnp.where(wid < rem, 1, 0)
        last = start + count - 1
        lane = lax.iota(jnp.int32, _L)
        steps = per + 2
        srcs = (wn_t, wz_t, wp_t)
        dsts = (on, oz, op)
        sets = ((s00, s01, s02, sem0), (s10, s11, s12, sem1))

        def issue(set_idx, gb):
            sl = pl.ds(gb * W128, W128)
            bufs = sets[set_idx]
            for t in range(3):
                pltpu.async_copy(srcs[t].at[:, sl], bufs[t], bufs[3])

        def drain(set_idx):
            bufs = sets[set_idx]
            for t in range(3):
                pltpu.make_async_copy(
                    srcs[t].at[:, pl.ds(0, W128)], bufs[t], bufs[3]).wait()

        def transpose_out(set_idx, gb):
            bufs = sets[set_idx]
            for t in range(3):
                slab = bufs[t]

                def tr_m(m, carry2):
                    p = m * _L + lane
                    row = lax.shift_right_logical(p, 1)
                    col0 = D * lax.bitwise_and(lane, 1)
                    for f in range(D):
                        v = slab[f, pl.ds(m * _L, _L)]
                        plsc.store_scatter(blk_v, [row, col0 + f], v)
                    return carry2

                lax.fori_loop(0, W128 // _L, tr_m, 0)
                pltpu.sync_copy(blk_v, dsts[t].at[pl.ds(gb * BR, BR)])

        issue(0, start)
        issue(1, jnp.minimum(start + 1, last))

        def step_body(i2, carry):
            for s in range(2):
                idx = i2 * 2 + s
                gb = start + jnp.minimum(idx, count - 1)
                gb_next = start + jnp.minimum(idx + 2, count - 1)
                drain(s)
                transpose_out(s, gb)
                issue(s, gb_next)
            return carry

        lax.fori_loop(0, (steps + 1) // 2, step_body, 0)
        drain(0)
        drain(1)

        Vt = V - G * 128
        if Vt:
            nrow = Vt * D // (2 * D)

            @pl.when(wid == NW - 1)
            def _tail():
                for src, dst in ((tn, on), (tz, oz), (tp, op)):
                    pltpu.sync_copy(src, dst.at[pl.ds(G * D, nrow)])

    return relayout


def _make_gather(N, V, D, C, NW):
    n_per_w = N // NW
    n_chunks = n_per_w // C
    mesh = plsc.VectorSubcoreMesh(core_axis_name="c", subcore_axis_name="s")

    @functools.partial(
        pl.kernel,
        out_type=jax.ShapeDtypeStruct((N, D), jnp.float32),
        mesh=mesh,
        scratch_types=[
            pltpu.VMEM((C,), jnp.int32),
            pltpu.VMEM((C,), jnp.float32),
            pltpu.VMEM((C,), jnp.float32),
            pltpu.VMEM((C,), jnp.float32),
            pltpu.VMEM((C, D), jnp.float32),
            pltpu.VMEM((C, D), jnp.float32),
            pltpu.VMEM((C, D), jnp.float32),
            pltpu.VMEM((C, D), jnp.float32),
            pltpu.SemaphoreType.DMA,
        ],
        compiler_params=pltpu.CompilerParams(use_tc_tiling_on_sc=False),
    )
    def sc_kernel(ids_hbm, wn_hbm, wz_hbm, wp_hbm, s0_hbm, s1_hbm, s2_hbm,
                  out_hbm, idx_v, p0_v, p1_v, p2_v, en_v, ez_v, ep_v, o_v,
                  sem):
        wid = lax.axis_index("s") * 2 + lax.axis_index("c")
        w_base = wid * n_per_w

        def chunk_body(c, carry):
            base = w_base + c * C
            pltpu.sync_copy(ids_hbm.at[pl.ds(base, C)], idx_v)
            cp0 = pltpu.async_copy(s0_hbm.at[idx_v], p0_v, sem)
            cp1 = pltpu.async_copy(s1_hbm.at[idx_v], p1_v, sem)
            cp2 = pltpu.async_copy(s2_hbm.at[idx_v], p2_v, sem)
            cp3 = pltpu.async_copy(wn_hbm.at[idx_v], en_v, sem)
            cp4 = pltpu.async_copy(wz_hbm.at[idx_v], ez_v, sem)
            cp5 = pltpu.async_copy(wp_hbm.at[idx_v], ep_v, sem)
            cp0.wait()
            cp1.wait()
            cp2.wait()
            cp3.wait()
            cp4.wait()
            cp5.wait()

            for i in range(C // _L):
                sl = pl.ds(i * _L, _L)
                l0 = p0_v[sl]
                l1 = p1_v[sl]
                l2 = p2_v[sl]
                m = jnp.maximum(jnp.maximum(l0, l1), l2)
                e0 = jnp.exp(l0 - m)
                e1 = jnp.exp(l1 - m)
                e2 = jnp.exp(l2 - m)
                inv = 1.0 / (e0 + e1 + e2)
                p0_v[sl] = e0 * inv
                p1_v[sl] = e1 * inv
                p2_v[sl] = e2 * inv

            def grp_body(g, carry):
                gsl = pl.ds(g * _L, _L)
                pv0 = p0_v[gsl]
                pv1 = p1_v[gsl]
                pv2 = p2_v[gsl]
                for t in range(_L):
                    j = g * _L + t
                    ts = jnp.full((_L,), t, jnp.int32)
                    pb0 = _bcast(pv0, ts)
                    pb1 = _bcast(pv1, ts)
                    pb2 = _bcast(pv2, ts)
                    for d in range(D // _L):
                        sl = pl.ds(d * _L, _L)
                        o_v[j, sl] = (pb0 * en_v[j, sl] + pb1 * ez_v[j, sl]
                                      + pb2 * ep_v[j, sl])
                return carry

            lax.fori_loop(0, C // _L, grp_body, 0)
            pltpu.sync_copy(o_v, out_hbm.at[pl.ds(base, C)])
            return carry

        lax.fori_loop(0, n_chunks, chunk_body, 0)

    return sc_kernel


@jax.jit
def kernel(input_ids, W_neg1, W_zero, W_pos1, sup_w):
    B, S = input_ids.shape
    V, D = W_neg1.shape
    N = B * S
    NW = 32
    C = 128
    ids_flat = input_ids.reshape(N).astype(jnp.int32)
    s0, s1, s2 = [sup_w[:, i] for i in range(3)]
    wn_t = W_neg1.T
    wz_t = W_zero.T
    wp_t = W_pos1.T
    Gfull = (V // 128) * 128
    nrow = (V - Gfull) * D // (2 * D)
    tn = W_neg1[Gfull:].reshape(nrow, 2 * D)
    tz = W_zero[Gfull:].reshape(nrow, 2 * D)
    tp = W_pos1[Gfull:].reshape(nrow, 2 * D)
    rn, rz, rp = _make_relayout(V, D, NW)(wn_t, wz_t, wp_t, tn, tz, tp)
    wn = rn.reshape(V, D)
    wz = rz.reshape(V, D)
    wp = rp.reshape(V, D)
    out = _make_gather(N, V, D, C, NW)(ids_flat, wn, wz, wp, s0, s1, s2)
    return out.reshape(B, S, D)


# speedup vs baseline: 2.0200x; 1.7607x over previous
import functools

import jax
import jax.numpy as jnp
from jax import lax
from jax.experimental import pallas as pl
from jax.experimental.pallas import tpu as pltpu
from jax.experimental.pallas import tpu_sc as plsc

_L = 16


def _bcast(vec, idx):
    dnums = lax.GatherDimensionNumbers(
        offset_dims=(), collapsed_slice_dims=(0,), start_index_map=(0,))
    return lax.gather(vec, idx[:, None], dnums, slice_sizes=(1,),
                      mode=lax.GatherScatterMode.PROMISE_IN_BOUNDS)


def _make_relayout(V, D, NW):
    G = V // 128
    K = 2
    W128 = K * 128
    BR = K * D
    B = G // K
    mesh = plsc.VectorSubcoreMesh(core_axis_name="c", subcore_axis_name="s")
    out_t = jax.ShapeDtypeStruct((V // 2, 2 * D), jnp.float32)
    slab_t = pltpu.VMEM((D, W128), jnp.float32)

    @functools.partial(
        pl.kernel,
        out_type=(out_t, out_t, out_t),
        mesh=mesh,
        scratch_types=[
            slab_t, slab_t, slab_t,
            slab_t, slab_t, slab_t,
            pltpu.VMEM((BR, 2 * D), jnp.float32),
            pltpu.SemaphoreType.DMA,
            pltpu.SemaphoreType.DMA,
        ],
        compiler_params=pltpu.CompilerParams(
            use_tc_tiling_on_sc=True, needs_layout_passes=False),
    )
    def relayout(wn_t, wz_t, wp_t, tn, tz, tp, on, oz, op,
                 s00, s01, s02, s10, s11, s12, blk_v, sem0, sem1):
        wid = lax.axis_index("s") * 2 + lax.axis_index("c")
        per = B // NW
        rem = B - per * NW
        start = wid * per + jnp.minimum(wid, rem)
        count = per + jnp.where(wid < rem, 1, 0)
        last = start + count - 1
        lane = lax.iota(jnp.int32, _L)
        steps = per + 2
        srcs = (wn_t, wz_t, wp_t)
        dsts = (on, oz, op)
        sets = ((s00, s01, s02, sem0), (s10, s11, s12, sem1))

        def issue(set_idx, gb):
            sl = pl.ds(gb * W128, W128)
            bufs = sets[set_idx]
            for t in range(3):
                pltpu.async_copy(srcs[t].at[:, sl], bufs[t], bufs[3])

        def drain(set_idx):
            bufs = sets[set_idx]
            for t in range(3):
                pltpu.make_async_copy(
                    srcs[t].at[:, pl.ds(0, W128)], bufs[t], bufs[3]).wait()

        stage_consts = []
        for dd in (1, 2, 4, 8):
            md = 1.0 - lax.bitwise_and(
                lax.shift_right_logical(lane, dd.bit_length() - 1), 1
            ).astype(jnp.float32)
            xd = lax.bitwise_xor(lane, dd)
            stage_consts.append((md, xd))

        def transpose_out(set_idx, gb):
            bufs = sets[set_idx]
            for t in range(3):
                slab = bufs[t]

                def tr_m(m, carry2):
                    for f0 in range(0, D, _L):
                        v = [slab[f0 + i, pl.ds(m * _L, _L)]
                             for i in range(_L)]
                        for si, dd in enumerate((1, 2, 4, 8)):
                            md, xd = stage_consts[si]
                            for i in range(_L):
                                if i & dd:
                                    continue
                                a, b = v[i], v[i + dd]
                                ta = _bcast(a, xd)
                                tb = _bcast(b, xd)
                                v[i] = md * (a - tb) + tb
                                v[i + dd] = md * (ta - b) + b
                        for j in range(_L):
                            row = 8 * m + (j >> 1)
                            col = D * (j & 1) + f0
                            blk_v[row, pl.ds(col, _L)] = v[j]
                    return carry2

                lax.fori_loop(0, W128 // _L, tr_m, 0)
                pltpu.sync_copy(blk_v, dsts[t].at[pl.ds(gb * BR, BR)])

        issue(0, start)
        issue(1, jnp.minimum(start + 1, last))

        def step_body(i2, carry):
            for s in range(2):
                idx = i2 * 2 + s
                gb = start + jnp.minimum(idx, count - 1)
                gb_next = start + jnp.minimum(idx + 2, count - 1)
                drain(s)
                transpose_out(s, gb)
                issue(s, gb_next)
            return carry

        lax.fori_loop(0, (steps + 1) // 2, step_body, 0)
        drain(0)
        drain(1)

        Vt = V - G * 128
        if Vt:
            nrow = Vt * D // (2 * D)

            @pl.when(wid == NW - 1)
            def _tail():
                for src, dst in ((tn, on), (tz, oz), (tp, op)):
                    pltpu.sync_copy(src, dst.at[pl.ds(G * D, nrow)])

    return relayout


def _make_gather(N, V, D, C, NW):
    n_per_w = N // NW
    n_chunks = n_per_w // C
    mesh = plsc.VectorSubcoreMesh(core_axis_name="c", subcore_axis_name="s")

    @functools.partial(
        pl.kernel,
        out_type=jax.ShapeDtypeStruct((N, D), jnp.float32),
        mesh=mesh,
        scratch_types=[
            pltpu.VMEM((C,), jnp.int32),
            pltpu.VMEM((C,), jnp.float32),
            pltpu.VMEM((C,), jnp.float32),
            pltpu.VMEM((C,), jnp.float32),
            pltpu.VMEM((C, D), jnp.float32),
            pltpu.VMEM((C, D), jnp.float32),
            pltpu.VMEM((C, D), jnp.float32),
            pltpu.VMEM((C, D), jnp.float32),
            pltpu.SemaphoreType.DMA,
        ],
        compiler_params=pltpu.CompilerParams(use_tc_tiling_on_sc=False),
    )
    def sc_kernel(ids_hbm, wn_hbm, wz_hbm, wp_hbm, s0_hbm, s1_hbm, s2_hbm,
                  out_hbm, idx_v, p0_v, p1_v, p2_v, en_v, ez_v, ep_v, o_v,
                  sem):
        wid = lax.axis_index("s") * 2 + lax.axis_index("c")
        w_base = wid * n_per_w

        def chunk_body(c, carry):
            base = w_base + c * C
            pltpu.sync_copy(ids_hbm.at[pl.ds(base, C)], idx_v)
            cp0 = pltpu.async_copy(s0_hbm.at[idx_v], p0_v, sem)
            cp1 = pltpu.async_copy(s1_hbm.at[idx_v], p1_v, sem)
            cp2 = pltpu.async_copy(s2_hbm.at[idx_v], p2_v, sem)
            cp3 = pltpu.async_copy(wn_hbm.at[idx_v], en_v, sem)
            cp4 = pltpu.async_copy(wz_hbm.at[idx_v], ez_v, sem)
            cp5 = pltpu.async_copy(wp_hbm.at[idx_v], ep_v, sem)
            cp0.wait()
            cp1.wait()
            cp2.wait()
            cp3.wait()
            cp4.wait()
            cp5.wait()

            for i in range(C // _L):
                sl = pl.ds(i * _L, _L)
                l0 = p0_v[sl]
                l1 = p1_v[sl]
                l2 = p2_v[sl]
                m = jnp.maximum(jnp.maximum(l0, l1), l2)
                e0 = jnp.exp(l0 - m)
                e1 = jnp.exp(l1 - m)
                e2 = jnp.exp(l2 - m)
                inv = 1.0 / (e0 + e1 + e2)
                p0_v[sl] = e0 * inv
                p1_v[sl] = e1 * inv
                p2_v[sl] = e2 * inv

            def grp_body(g, carry):
                gsl = pl.ds(g * _L, _L)
                pv0 = p0_v[gsl]
                pv1 = p1_v[gsl]
                pv2 = p2_v[gsl]
                for t in range(_L):
                    j = g * _L + t
                    ts = jnp.full((_L,), t, jnp.int32)
                    pb0 = _bcast(pv0, ts)
                    pb1 = _bcast(pv1, ts)
                    pb2 = _bcast(pv2, ts)
                    for d in range(D // _L):
                        sl = pl.ds(d * _L, _L)
                        o_v[j, sl] = (pb0 * en_v[j, sl] + pb1 * ez_v[j, sl]
                                      + pb2 * ep_v[j, sl])
                return carry

            lax.fori_loop(0, C // _L, grp_body, 0)
            pltpu.sync_copy(o_v, out_hbm.at[pl.ds(base, C)])
            return carry

        lax.fori_loop(0, n_chunks, chunk_body, 0)

    return sc_kernel


@jax.jit
def kernel(input_ids, W_neg1, W_zero, W_pos1, sup_w):
    B, S = input_ids.shape
    V, D = W_neg1.shape
    N = B * S
    NW = 32
    C = 128
    ids_flat = input_ids.reshape(N).astype(jnp.int32)
    s0, s1, s2 = [sup_w[:, i] for i in range(3)]
    wn_t = W_neg1.T
    wz_t = W_zero.T
    wp_t = W_pos1.T
    Gfull = (V // 128) * 128
    nrow = (V - Gfull) * D // (2 * D)
    tn = W_neg1[Gfull:].reshape(nrow, 2 * D)
    tz = W_zero[Gfull:].reshape(nrow, 2 * D)
    tp = W_pos1[Gfull:].reshape(nrow, 2 * D)
    rn, rz, rp = _make_relayout(V, D, NW)(wn_t, wz_t, wp_t, tn, tz, tp)
    wn = rn.reshape(V, D)
    wz = rz.reshape(V, D)
    wp = rp.reshape(V, D)
    out = _make_gather(N, V, D, C, NW)(ids_flat, wn, wz, wp, s0, s1, s2)
    return out.reshape(B, S, D)


# speedup vs baseline: 2.3250x; 1.1510x over previous
import functools

import jax
import jax.numpy as jnp
from jax import lax
from jax.experimental import pallas as pl
from jax.experimental.pallas import tpu as pltpu
from jax.experimental.pallas import tpu_sc as plsc

_L = 16


def _bcast(vec, idx):
    dnums = lax.GatherDimensionNumbers(
        offset_dims=(), collapsed_slice_dims=(0,), start_index_map=(0,))
    return lax.gather(vec, idx[:, None], dnums, slice_sizes=(1,),
                      mode=lax.GatherScatterMode.PROMISE_IN_BOUNDS)


def _make_relayout(V, D, NW):
    G = V // 128
    K = 2
    W128 = K * 128
    BR = K * D
    B = G // K
    mesh = plsc.VectorSubcoreMesh(core_axis_name="c", subcore_axis_name="s")
    out_t = jax.ShapeDtypeStruct((V // 2, 2 * D), jnp.float32)
    slab_t = pltpu.VMEM((D, W128), jnp.float32)

    @functools.partial(
        pl.kernel,
        out_type=(out_t, out_t, out_t),
        mesh=mesh,
        scratch_types=[
            slab_t, slab_t, slab_t,
            slab_t, slab_t, slab_t,
            pltpu.VMEM((BR, 2 * D), jnp.float32),
            pltpu.SemaphoreType.DMA,
            pltpu.SemaphoreType.DMA,
        ],
        compiler_params=pltpu.CompilerParams(
            use_tc_tiling_on_sc=True, needs_layout_passes=False),
    )
    def relayout(wn_t, wz_t, wp_t, tn, tz, tp, on, oz, op,
                 s00, s01, s02, s10, s11, s12, blk_v, sem0, sem1):
        wid = lax.axis_index("s") * 2 + lax.axis_index("c")
        per = B // NW
        rem = B - per * NW
        start = wid * per + jnp.minimum(wid, rem)
        count = per + jnp.where(wid < rem, 1, 0)
        last = start + count - 1
        lane = lax.iota(jnp.int32, _L)
        steps = per + 2
        srcs = (wn_t, wz_t, wp_t)
        dsts = (on, oz, op)
        sets = ((s00, s01, s02, sem0), (s10, s11, s12, sem1))

        def issue(set_idx, gb):
            sl = pl.ds(gb * W128, W128)
            bufs = sets[set_idx]
            for t in range(3):
                pltpu.async_copy(srcs[t].at[:, sl], bufs[t], bufs[3])

        def drain(set_idx):
            bufs = sets[set_idx]
            for t in range(3):
                pltpu.make_async_copy(
                    srcs[t].at[:, pl.ds(0, W128)], bufs[t], bufs[3]).wait()

        stage_consts = []
        for dd in (1, 2, 4, 8):
            md = 1.0 - lax.bitwise_and(
                lax.shift_right_logical(lane, dd.bit_length() - 1), 1
            ).astype(jnp.float32)
            xd = lax.bitwise_xor(lane, dd)
            stage_consts.append((md, xd))

        def transpose_out(set_idx, gb):
            bufs = sets[set_idx]
            for t in range(3):
                slab = bufs[t]

                def tr_m(m, carry2):
                    for f0 in range(0, D, _L):
                        v = [slab[f0 + i, pl.ds(m * _L, _L)]
                             for i in range(_L)]
                        for si, dd in enumerate((1, 2, 4, 8)):
                            md, xd = stage_consts[si]
                            for i in range(_L):
                                if i & dd:
                                    continue
                                a, b = v[i], v[i + dd]
                                ta = _bcast(a, xd)
                                tb = _bcast(b, xd)
                                v[i] = md * (a - tb) + tb
                                v[i + dd] = md * (ta - b) + b
                        for j in range(_L):
                            row = 8 * m + (j >> 1)
                            col = D * (j & 1) + f0
                            blk_v[row, pl.ds(col, _L)] = v[j]
                    return carry2

                lax.fori_loop(0, W128 // _L, tr_m, 0)
                pltpu.sync_copy(blk_v, dsts[t].at[pl.ds(gb * BR, BR)])

        issue(0, start)
        issue(1, jnp.minimum(start + 1, last))

        def step_body(i2, carry):
            for s in range(2):
                idx = i2 * 2 + s
                gb = start + jnp.minimum(idx, count - 1)
                gb_next = start + jnp.minimum(idx + 2, count - 1)
                drain(s)
                transpose_out(s, gb)
                issue(s, gb_next)
            return carry

        lax.fori_loop(0, (steps + 1) // 2, step_body, 0)
        drain(0)
        drain(1)

        Vt = V - G * 128
        if Vt:
            nrow = Vt * D // (2 * D)

            @pl.when(wid == NW - 1)
            def _tail():
                for src, dst in ((tn, on), (tz, oz), (tp, op)):
                    pltpu.sync_copy(src, dst.at[pl.ds(G * D, nrow)])

    return relayout


def _make_gather(N, V, D, C, NW):
    n_per_w = N // NW
    n_chunks = n_per_w // C
    mesh = plsc.VectorSubcoreMesh(core_axis_name="c", subcore_axis_name="s")

    buf_set = [
        pltpu.VMEM((C,), jnp.int32),
        pltpu.VMEM((C,), jnp.float32),
        pltpu.VMEM((C,), jnp.float32),
        pltpu.VMEM((C,), jnp.float32),
        pltpu.VMEM((C, D), jnp.float32),
        pltpu.VMEM((C, D), jnp.float32),
        pltpu.VMEM((C, D), jnp.float32),
        pltpu.SemaphoreType.DMA,
    ]

    @functools.partial(
        pl.kernel,
        out_type=jax.ShapeDtypeStruct((N, D), jnp.float32),
        mesh=mesh,
        scratch_types=buf_set + buf_set + [
            pltpu.VMEM((C, D), jnp.float32),
        ],
        compiler_params=pltpu.CompilerParams(use_tc_tiling_on_sc=False),
    )
    def sc_kernel(ids_hbm, wn_hbm, wz_hbm, wp_hbm, s0_hbm, s1_hbm, s2_hbm,
                  out_hbm,
                  idx_v0, p0_v0, p1_v0, p2_v0, en_v0, ez_v0, ep_v0, sem0,
                  idx_v1, p0_v1, p1_v1, p2_v1, en_v1, ez_v1, ep_v1, sem1,
                  o_v):
        wid = lax.axis_index("s") * 2 + lax.axis_index("c")
        w_base = wid * n_per_w
        sets = ((idx_v0, p0_v0, p1_v0, p2_v0, en_v0, ez_v0, ep_v0, sem0),
                (idx_v1, p0_v1, p1_v1, p2_v1, en_v1, ez_v1, ep_v1, sem1))
        tabs = (s0_hbm, s1_hbm, s2_hbm, wn_hbm, wz_hbm, wp_hbm)

        def issue(s, c):
            idx_v, p0_v, p1_v, p2_v, en_v, ez_v, ep_v, sem = sets[s]
            pltpu.sync_copy(ids_hbm.at[pl.ds(w_base + c * C, C)], idx_v)
            for src, dst in zip(tabs, (p0_v, p1_v, p2_v, en_v, ez_v, ep_v)):
                pltpu.async_copy(src.at[idx_v], dst, sem)

        def drain(s):
            idx_v, p0_v, p1_v, p2_v, en_v, ez_v, ep_v, sem = sets[s]
            for src, dst in zip(tabs, (p0_v, p1_v, p2_v, en_v, ez_v, ep_v)):
                pltpu.make_async_copy(src.at[idx_v], dst, sem).wait()

        def compute(s, c):
            idx_v, p0_v, p1_v, p2_v, en_v, ez_v, ep_v, sem = sets[s]
            for i in range(C // _L):
                sl = pl.ds(i * _L, _L)
                l0 = p0_v[sl]
                l1 = p1_v[sl]
                l2 = p2_v[sl]
                m = jnp.maximum(jnp.maximum(l0, l1), l2)
                e0 = jnp.exp(l0 - m)
                e1 = jnp.exp(l1 - m)
                e2 = jnp.exp(l2 - m)
                inv = 1.0 / (e0 + e1 + e2)
                p0_v[sl] = e0 * inv
                p1_v[sl] = e1 * inv
                p2_v[sl] = e2 * inv

            def grp_body(g, carry):
                gsl = pl.ds(g * _L, _L)
                pv0 = p0_v[gsl]
                pv1 = p1_v[gsl]
                pv2 = p2_v[gsl]
                for t in range(_L):
                    j = g * _L + t
                    ts = jnp.full((_L,), t, jnp.int32)
                    pb0 = _bcast(pv0, ts)
                    pb1 = _bcast(pv1, ts)
                    pb2 = _bcast(pv2, ts)
                    for d in range(D // _L):
                        sl = pl.ds(d * _L, _L)
                        o_v[j, sl] = (pb0 * en_v[j, sl] + pb1 * ez_v[j, sl]
                                      + pb2 * ep_v[j, sl])
                return carry

            lax.fori_loop(0, C // _L, grp_body, 0)
            pltpu.sync_copy(o_v, out_hbm.at[pl.ds(w_base + c * C, C)])

        issue(0, 0)
        issue(1, 1)

        def pair_body(i2, carry):
            for s in range(2):
                c = i2 * 2 + s
                drain(s)
                compute(s, c)
                issue(s, jnp.minimum(c + 2, n_chunks - 1))
            return carry

        lax.fori_loop(0, n_chunks // 2, pair_body, 0)
        drain(0)
        drain(1)

    return sc_kernel


@jax.jit
def kernel(input_ids, W_neg1, W_zero, W_pos1, sup_w):
    B, S = input_ids.shape
    V, D = W_neg1.shape
    N = B * S
    NW = 32
    C = 128
    ids_flat = input_ids.reshape(N).astype(jnp.int32)
    s0, s1, s2 = [sup_w[:, i] for i in range(3)]
    wn_t = W_neg1.T
    wz_t = W_zero.T
    wp_t = W_pos1.T
    Gfull = (V // 128) * 128
    nrow = (V - Gfull) * D // (2 * D)
    tn = W_neg1[Gfull:].reshape(nrow, 2 * D)
    tz = W_zero[Gfull:].reshape(nrow, 2 * D)
    tp = W_pos1[Gfull:].reshape(nrow, 2 * D)
    rn, rz, rp = _make_relayout(V, D, NW)(wn_t, wz_t, wp_t, tn, tz, tp)
    wn = rn.reshape(V, D)
    wz = rz.reshape(V, D)
    wp = rp.reshape(V, D)
    out = _make_gather(N, V, D, C, NW)(ids_flat, wn, wz, wp, s0, s1, s2)
    return out.reshape(B, S, D)


# speedup vs baseline: 2.9040x; 1.2491x over previous
import functools

import jax
import jax.numpy as jnp
from jax import lax
from jax.experimental import pallas as pl
from jax.experimental.pallas import tpu as pltpu
from jax.experimental.pallas import tpu_sc as plsc

_L = 16


def _bcast(vec, idx):
    dnums = lax.GatherDimensionNumbers(
        offset_dims=(), collapsed_slice_dims=(0,), start_index_map=(0,))
    return lax.gather(vec, idx[:, None], dnums, slice_sizes=(1,),
                      mode=lax.GatherScatterMode.PROMISE_IN_BOUNDS)


def _make_relayout(V, D, NW):
    G = V // 128
    K = 2
    W128 = K * 128
    BR = K * D
    B = G // K
    mesh = plsc.VectorSubcoreMesh(core_axis_name="c", subcore_axis_name="s")
    out_t = jax.ShapeDtypeStruct((V // 2, 2 * D), jnp.float32)
    slab_t = pltpu.VMEM((D, W128), jnp.float32)

    @functools.partial(
        pl.kernel,
        out_type=(out_t, out_t, out_t),
        mesh=mesh,
        scratch_types=[
            slab_t, slab_t, slab_t,
            slab_t, slab_t, slab_t,
            pltpu.VMEM((BR, 2 * D), jnp.float32),
            pltpu.SemaphoreType.DMA,
            pltpu.SemaphoreType.DMA,
        ],
        compiler_params=pltpu.CompilerParams(
            use_tc_tiling_on_sc=True, needs_layout_passes=False),
    )
    def relayout(wn_t, wz_t, wp_t, tn, tz, tp, on, oz, op,
                 s00, s01, s02, s10, s11, s12, blk_v, sem0, sem1):
        wid = lax.axis_index("s") * 2 + lax.axis_index("c")
        per = B // NW
        rem = B - per * NW
        start = wid * per + jnp.minimum(wid, rem)
        count = per + jnp.where(wid < rem, 1, 0)
        last = start + count - 1
        lane = lax.iota(jnp.int32, _L)
        steps = per + 2
        srcs = (wn_t, wz_t, wp_t)
        dsts = (on, oz, op)
        sets = ((s00, s01, s02, sem0), (s10, s11, s12, sem1))

        def issue(set_idx, gb):
            sl = pl.ds(gb * W128, W128)
            bufs = sets[set_idx]
            for t in range(3):
                pltpu.async_copy(srcs[t].at[:, sl], bufs[t], bufs[3])

        def drain(set_idx):
            bufs = sets[set_idx]
            for t in range(3):
                pltpu.make_async_copy(
                    srcs[t].at[:, pl.ds(0, W128)], bufs[t], bufs[3]).wait()

        stage_consts = []
        for dd in (1, 2, 4, 8):
            md = lax.bitwise_and(lane, dd) == 0
            xd = lax.bitwise_xor(lane, dd)
            stage_consts.append((md, xd))

        def transpose_out(set_idx, gb):
            bufs = sets[set_idx]
            for t in range(3):
                slab = bufs[t]

                def tr_m(m, carry2):
                    for f0 in range(0, D, _L):
                        v = [slab[f0 + i, pl.ds(m * _L, _L)]
                             for i in range(_L)]
                        for si, dd in enumerate((1, 2, 4, 8)):
                            md, xd = stage_consts[si]
                            for i in range(_L):
                                if i & dd:
                                    continue
                                a, b = v[i], v[i + dd]
                                ta = _bcast(a, xd)
                                tb = _bcast(b, xd)
                                v[i] = jnp.where(md, a, tb)
                                v[i + dd] = jnp.where(md, ta, b)
                        for j in range(_L):
                            row = 8 * m + (j >> 1)
                            col = D * (j & 1) + f0
                            blk_v[row, pl.ds(col, _L)] = v[j]
                    return carry2

                lax.fori_loop(0, W128 // _L, tr_m, 0)
                pltpu.sync_copy(blk_v, dsts[t].at[pl.ds(gb * BR, BR)])

        issue(0, start)
        issue(1, jnp.minimum(start + 1, last))

        def step_body(i2, carry):
            for s in range(2):
                idx = i2 * 2 + s
                gb = start + jnp.minimum(idx, count - 1)
                gb_next = start + jnp.minimum(idx + 2, count - 1)
                drain(s)
                transpose_out(s, gb)
                issue(s, gb_next)
            return carry

        lax.fori_loop(0, (steps + 1) // 2, step_body, 0)
        drain(0)
        drain(1)

        Vt = V - G * 128
        if Vt:
            nrow = Vt * D // (2 * D)

            @pl.when(wid == NW - 1)
            def _tail():
                for src, dst in ((tn, on), (tz, oz), (tp, op)):
                    pltpu.sync_copy(src, dst.at[pl.ds(G * D, nrow)])

    return relayout


def _make_gather(N, V, D, C, NW):
    n_per_w = N // NW
    n_chunks = n_per_w // C
    mesh = plsc.VectorSubcoreMesh(core_axis_name="c", subcore_axis_name="s")

    buf_set = [
        pltpu.VMEM((C,), jnp.int32),
        pltpu.VMEM((C,), jnp.float32),
        pltpu.VMEM((C,), jnp.float32),
        pltpu.VMEM((C,), jnp.float32),
        pltpu.VMEM((C, D), jnp.float32),
        pltpu.VMEM((C, D), jnp.float32),
        pltpu.VMEM((C, D), jnp.float32),
        pltpu.SemaphoreType.DMA,
    ]

    @functools.partial(
        pl.kernel,
        out_type=jax.ShapeDtypeStruct((N, D), jnp.float32),
        mesh=mesh,
        scratch_types=buf_set + buf_set + [
            pltpu.VMEM((C, D), jnp.float32),
        ],
        compiler_params=pltpu.CompilerParams(use_tc_tiling_on_sc=False),
    )
    def sc_kernel(ids_hbm, wn_hbm, wz_hbm, wp_hbm, s0_hbm, s1_hbm, s2_hbm,
                  out_hbm,
                  idx_v0, p0_v0, p1_v0, p2_v0, en_v0, ez_v0, ep_v0, sem0,
                  idx_v1, p0_v1, p1_v1, p2_v1, en_v1, ez_v1, ep_v1, sem1,
                  o_v):
        wid = lax.axis_index("s") * 2 + lax.axis_index("c")
        w_base = wid * n_per_w
        sets = ((idx_v0, p0_v0, p1_v0, p2_v0, en_v0, ez_v0, ep_v0, sem0),
                (idx_v1, p0_v1, p1_v1, p2_v1, en_v1, ez_v1, ep_v1, sem1))
        tabs = (s0_hbm, s1_hbm, s2_hbm, wn_hbm, wz_hbm, wp_hbm)

        def issue(s, c):
            idx_v, p0_v, p1_v, p2_v, en_v, ez_v, ep_v, sem = sets[s]
            pltpu.sync_copy(ids_hbm.at[pl.ds(w_base + c * C, C)], idx_v)
            for src, dst in zip(tabs, (p0_v, p1_v, p2_v, en_v, ez_v, ep_v)):
                pltpu.async_copy(src.at[idx_v], dst, sem)

        def drain(s):
            idx_v, p0_v, p1_v, p2_v, en_v, ez_v, ep_v, sem = sets[s]
            for src, dst in zip(tabs, (p0_v, p1_v, p2_v, en_v, ez_v, ep_v)):
                pltpu.make_async_copy(src.at[idx_v], dst, sem).wait()

        def compute(s, c):
            idx_v, p0_v, p1_v, p2_v, en_v, ez_v, ep_v, sem = sets[s]
            for i in range(C // _L):
                sl = pl.ds(i * _L, _L)
                l0 = p0_v[sl]
                l1 = p1_v[sl]
                l2 = p2_v[sl]
                m = jnp.maximum(jnp.maximum(l0, l1), l2)
                e0 = jnp.exp(l0 - m)
                e1 = jnp.exp(l1 - m)
                e2 = jnp.exp(l2 - m)
                inv = 1.0 / (e0 + e1 + e2)
                p0_v[sl] = e0 * inv
                p1_v[sl] = e1 * inv
                p2_v[sl] = e2 * inv

            def grp_body(g, carry):
                gsl = pl.ds(g * _L, _L)
                pv0 = p0_v[gsl]
                pv1 = p1_v[gsl]
                pv2 = p2_v[gsl]
                for t in range(_L):
                    j = g * _L + t
                    ts = jnp.full((_L,), t, jnp.int32)
                    pb0 = _bcast(pv0, ts)
                    pb1 = _bcast(pv1, ts)
                    pb2 = _bcast(pv2, ts)
                    for d in range(D // _L):
                        sl = pl.ds(d * _L, _L)
                        o_v[j, sl] = (pb0 * en_v[j, sl] + pb1 * ez_v[j, sl]
                                      + pb2 * ep_v[j, sl])
                return carry

            lax.fori_loop(0, C // _L, grp_body, 0)
            pltpu.sync_copy(o_v, out_hbm.at[pl.ds(w_base + c * C, C)])

        issue(0, 0)
        issue(1, 1)

        def pair_body(i2, carry):
            for s in range(2):
                c = i2 * 2 + s
                drain(s)
                compute(s, c)
                issue(s, jnp.minimum(c + 2, n_chunks - 1))
            return carry

        lax.fori_loop(0, n_chunks // 2, pair_body, 0)
        drain(0)
        drain(1)

    return sc_kernel


@jax.jit
def kernel(input_ids, W_neg1, W_zero, W_pos1, sup_w):
    B, S = input_ids.shape
    V, D = W_neg1.shape
    N = B * S
    NW = 32
    C = 128
    ids_flat = input_ids.reshape(N).astype(jnp.int32)
    s0, s1, s2 = [sup_w[:, i] for i in range(3)]
    wn_t = W_neg1.T
    wz_t = W_zero.T
    wp_t = W_pos1.T
    Gfull = (V // 128) * 128
    nrow = (V - Gfull) * D // (2 * D)
    tn = W_neg1[Gfull:].reshape(nrow, 2 * D)
    tz = W_zero[Gfull:].reshape(nrow, 2 * D)
    tp = W_pos1[Gfull:].reshape(nrow, 2 * D)
    rn, rz, rp = _make_relayout(V, D, NW)(wn_t, wz_t, wp_t, tn, tz, tp)
    wn = rn.reshape(V, D)
    wz = rz.reshape(V, D)
    wp = rp.reshape(V, D)
    out = _make_gather(N, V, D, C, NW)(ids_flat, wn, wz, wp, s0, s1, s2)
    return out.reshape(B, S, D)


# speedup vs baseline: 3.2481x; 1.1185x over previous
import functools

import jax
import jax.numpy as jnp
from jax import lax
from jax.experimental import pallas as pl
from jax.experimental.pallas import tpu as pltpu
from jax.experimental.pallas import tpu_sc as plsc

_L = 16


def _bcast(vec, idx):
    dnums = lax.GatherDimensionNumbers(
        offset_dims=(), collapsed_slice_dims=(0,), start_index_map=(0,))
    return lax.gather(vec, idx[:, None], dnums, slice_sizes=(1,),
                      mode=lax.GatherScatterMode.PROMISE_IN_BOUNDS)


def _make_relayout(V, D, NW):
    G = V // 128
    K = 2
    W128 = K * 128
    BR = K * D
    B = G // K
    mesh = plsc.VectorSubcoreMesh(core_axis_name="c", subcore_axis_name="s")
    out_t = jax.ShapeDtypeStruct((V // 2, 2 * D), jnp.float32)
    slab_t = pltpu.VMEM((D, W128), jnp.float32)

    @functools.partial(
        pl.kernel,
        out_type=(out_t, out_t, out_t,
                  jax.ShapeDtypeStruct((K * D // 2, 2 * D), jnp.float32)),
        mesh=mesh,
        scratch_types=[
            slab_t, slab_t, slab_t,
            slab_t, slab_t, slab_t,
            pltpu.VMEM((BR // 2, 2 * D), jnp.float32),
            pltpu.VMEM((BR // 2, 2 * D), jnp.float32),
            pltpu.SemaphoreType.DMA,
            pltpu.SemaphoreType.DMA,
            pltpu.SemaphoreType.DMA,
            pltpu.SemaphoreType.DMA,
        ],
        compiler_params=pltpu.CompilerParams(
            use_tc_tiling_on_sc=True, needs_layout_passes=False),
    )
    def relayout(wn_t, wz_t, wp_t, tn, tz, tp, on, oz, op, dump,
                 s00, s01, s02, s10, s11, s12, blk_a, blk_b, sem0, sem1,
                 sem_oa, sem_ob):
        wid = lax.axis_index("s") * 2 + lax.axis_index("c")
        per = B // NW
        rem = B - per * NW
        start = wid * per + jnp.minimum(wid, rem)
        count = per + jnp.where(wid < rem, 1, 0)
        last = start + count - 1
        lane = lax.iota(jnp.int32, _L)
        steps = per + 2
        srcs = (wn_t, wz_t, wp_t)
        dsts = (on, oz, op)
        sets = ((s00, s01, s02, sem0), (s10, s11, s12, sem1))

        def issue(set_idx, gb):
            sl = pl.ds(gb * W128, W128)
            bufs = sets[set_idx]
            for t in range(3):
                pltpu.async_copy(srcs[t].at[:, sl], bufs[t], bufs[3])

        def drain(set_idx):
            bufs = sets[set_idx]
            for t in range(3):
                pltpu.make_async_copy(
                    srcs[t].at[:, pl.ds(0, W128)], bufs[t], bufs[3]).wait()

        stage_consts = []
        for dd in (1, 2, 4, 8):
            md = lax.bitwise_and(lane, dd) == 0
            xd = lax.bitwise_xor(lane, dd)
            stage_consts.append((md, xd))

        HR = BR // 2
        halves = ((blk_a, sem_oa, 0), (blk_b, sem_ob, HR))

        def transpose_out(set_idx, gb):
            bufs = sets[set_idx]
            for t in range(3):
                slab = bufs[t]
                for blk, sem_o, roff in halves:
                    pltpu.make_async_copy(
                        blk, dsts[t].at[pl.ds(0, HR)], sem_o).wait()

                    def tr_m(m2, carry2):
                        for f0 in range(0, D, _L):
                            sl = pl.ds((m2 * _L) + roff * 2, _L)
                            v = [slab[f0 + i, sl] for i in range(_L)]
                            for si, dd in enumerate((1, 2, 4, 8)):
                                md, xd = stage_consts[si]
                                for i in range(_L):
                                    if i & dd:
                                        continue
                                    a, b = v[i], v[i + dd]
                                    ta = _bcast(a, xd)
                                    tb = _bcast(b, xd)
                                    v[i] = jnp.where(md, a, tb)
                                    v[i + dd] = jnp.where(md, ta, b)
                            for j in range(_L):
                                row = 8 * m2 + (j >> 1)
                                col = D * (j & 1) + f0
                                blk[row, pl.ds(col, _L)] = v[j]
                        return carry2

                    lax.fori_loop(0, HR // 8, tr_m, 0)
                    pltpu.async_copy(
                        blk, dsts[t].at[pl.ds(gb * BR + roff, HR)], sem_o)

        pltpu.async_copy(blk_a, dump, sem_oa)
        pltpu.async_copy(blk_b, dump, sem_ob)

        issue(0, start)
        issue(1, jnp.minimum(start + 1, last))

        def step_body(i2, carry):
            for s in range(2):
                idx = i2 * 2 + s
                gb = start + jnp.minimum(idx, count - 1)
                gb_next = start + jnp.minimum(idx + 2, count - 1)
                drain(s)
                transpose_out(s, gb)
                issue(s, gb_next)
            return carry

        lax.fori_loop(0, (steps + 1) // 2, step_body, 0)
        drain(0)
        drain(1)
        pltpu.make_async_copy(blk_a, on.at[pl.ds(0, BR // 2)], sem_oa).wait()
        pltpu.make_async_copy(blk_b, on.at[pl.ds(0, BR // 2)], sem_ob).wait()

        Vt = V - G * 128
        if Vt:
            nrow = Vt * D // (2 * D)

            @pl.when(wid == NW - 1)
            def _tail():
                for src, dst in ((tn, on), (tz, oz), (tp, op)):
                    pltpu.sync_copy(src, dst.at[pl.ds(G * D, nrow)])

    return relayout


def _make_gather(N, V, D, C, NW):
    n_per_w = N // NW
    n_chunks = n_per_w // C
    mesh = plsc.VectorSubcoreMesh(core_axis_name="c", subcore_axis_name="s")

    buf_set = [
        pltpu.VMEM((C,), jnp.int32),
        pltpu.VMEM((C,), jnp.float32),
        pltpu.VMEM((C,), jnp.float32),
        pltpu.VMEM((C,), jnp.float32),
        pltpu.VMEM((C, D), jnp.float32),
        pltpu.VMEM((C, D), jnp.float32),
        pltpu.VMEM((C, D), jnp.float32),
        pltpu.SemaphoreType.DMA,
    ]

    @functools.partial(
        pl.kernel,
        out_type=jax.ShapeDtypeStruct((N, D), jnp.float32),
        mesh=mesh,
        scratch_types=buf_set + buf_set + [
            pltpu.VMEM((C, D), jnp.float32),
        ],
        compiler_params=pltpu.CompilerParams(use_tc_tiling_on_sc=False),
    )
    def sc_kernel(ids_hbm, wn_hbm, wz_hbm, wp_hbm, s0_hbm, s1_hbm, s2_hbm,
                  out_hbm,
                  idx_v0, p0_v0, p1_v0, p2_v0, en_v0, ez_v0, ep_v0, sem0,
                  idx_v1, p0_v1, p1_v1, p2_v1, en_v1, ez_v1, ep_v1, sem1,
                  o_v):
        wid = lax.axis_index("s") * 2 + lax.axis_index("c")
        w_base = wid * n_per_w
        sets = ((idx_v0, p0_v0, p1_v0, p2_v0, en_v0, ez_v0, ep_v0, sem0),
                (idx_v1, p0_v1, p1_v1, p2_v1, en_v1, ez_v1, ep_v1, sem1))
        tabs = (s0_hbm, s1_hbm, s2_hbm, wn_hbm, wz_hbm, wp_hbm)

        def issue(s, c):
            idx_v, p0_v, p1_v, p2_v, en_v, ez_v, ep_v, sem = sets[s]
            pltpu.sync_copy(ids_hbm.at[pl.ds(w_base + c * C, C)], idx_v)
            for src, dst in zip(tabs, (p0_v, p1_v, p2_v, en_v, ez_v, ep_v)):
                pltpu.async_copy(src.at[idx_v], dst, sem)

        def drain(s):
            idx_v, p0_v, p1_v, p2_v, en_v, ez_v, ep_v, sem = sets[s]
            for src, dst in zip(tabs, (p0_v, p1_v, p2_v, en_v, ez_v, ep_v)):
                pltpu.make_async_copy(src.at[idx_v], dst, sem).wait()

        def compute(s, c):
            idx_v, p0_v, p1_v, p2_v, en_v, ez_v, ep_v, sem = sets[s]
            for i in range(C // _L):
                sl = pl.ds(i * _L, _L)
                l0 = p0_v[sl]
                l1 = p1_v[sl]
                l2 = p2_v[sl]
                m = jnp.maximum(jnp.maximum(l0, l1), l2)
                e0 = jnp.exp(l0 - m)
                e1 = jnp.exp(l1 - m)
                e2 = jnp.exp(l2 - m)
                inv = 1.0 / (e0 + e1 + e2)
                p0_v[sl] = e0 * inv
                p1_v[sl] = e1 * inv
                p2_v[sl] = e2 * inv

            def grp_body(g, carry):
                gsl = pl.ds(g * _L, _L)
                pv0 = p0_v[gsl]
                pv1 = p1_v[gsl]
                pv2 = p2_v[gsl]
                for t in range(_L):
                    j = g * _L + t
                    ts = jnp.full((_L,), t, jnp.int32)
                    pb0 = _bcast(pv0, ts)
                    pb1 = _bcast(pv1, ts)
                    pb2 = _bcast(pv2, ts)
                    for d in range(D // _L):
                        sl = pl.ds(d * _L, _L)
                        o_v[j, sl] = (pb0 * en_v[j, sl] + pb1 * ez_v[j, sl]
                                      + pb2 * ep_v[j, sl])
                return carry

            lax.fori_loop(0, C // _L, grp_body, 0)
            pltpu.sync_copy(o_v, out_hbm.at[pl.ds(w_base + c * C, C)])

        issue(0, 0)
        issue(1, 1)

        def pair_body(i2, carry):
            for s in range(2):
                c = i2 * 2 + s
                drain(s)
                compute(s, c)
                issue(s, jnp.minimum(c + 2, n_chunks - 1))
            return carry

        lax.fori_loop(0, n_chunks // 2, pair_body, 0)
        drain(0)
        drain(1)

    return sc_kernel


@jax.jit
def kernel(input_ids, W_neg1, W_zero, W_pos1, sup_w):
    B, S = input_ids.shape
    V, D = W_neg1.shape
    N = B * S
    NW = 32
    C = 128
    ids_flat = input_ids.reshape(N).astype(jnp.int32)
    s0, s1, s2 = [sup_w[:, i] for i in range(3)]
    wn_t = W_neg1.T
    wz_t = W_zero.T
    wp_t = W_pos1.T
    Gfull = (V // 128) * 128
    nrow = (V - Gfull) * D // (2 * D)
    tn = W_neg1[Gfull:].reshape(nrow, 2 * D)
    tz = W_zero[Gfull:].reshape(nrow, 2 * D)
    tp = W_pos1[Gfull:].reshape(nrow, 2 * D)
    rn, rz, rp, _dump = _make_relayout(V, D, NW)(wn_t, wz_t, wp_t, tn, tz, tp)
    wn = rn.reshape(V, D)
    wz = rz.reshape(V, D)
    wp = rp.reshape(V, D)
    out = _make_gather(N, V, D, C, NW)(ids_flat, wn, wz, wp, s0, s1, s2)
    return out.reshape(B, S, D)
